# baseline (speedup 1.0000x reference)
import jax
import jax.numpy as jnp
from jax.experimental import pallas as pl
from jax.experimental.pallas import tpu as pltpu

_BATCH = 1024
_N_CONT = 13
_N_CAT = 26
_VOCAB = 1000
_EMB = 50
_TOTAL_IN = _N_CONT + _N_CAT * _VOCAB
_TOTAL_OUT = _N_CONT + _N_CAT * _EMB
_TILE_B = 128
_WPAD = 1152

_STARTS = [_N_CONT + f * _VOCAB for f in range(_N_CAT)]
_ALIGNED = [(s // 128) * 128 for s in _STARTS]
_OFFS = [s - a for s, a in zip(_STARTS, _ALIGNED)]


def _body(x_ref, w_ref, o_ref):
    o_ref[:, :_N_CONT] = x_ref[:, :_N_CONT]
    for f in range(_N_CAT):
        a = _ALIGNED[f]
        w = min(_WPAD, _TOTAL_IN - a)
        x = x_ref[:, a : a + w]
        o_ref[:, _N_CONT + f * _EMB : _N_CONT + (f + 1) * _EMB] = jnp.dot(
            x,
            w_ref[f, :w, :],
            preferred_element_type=jnp.float32,
            precision=jax.lax.Precision.DEFAULT,
        )


def kernel(inputs, embeddings):
    offs = jnp.asarray(_OFFS, dtype=jnp.int32)
    wpad = jnp.pad(embeddings, ((0, 0), (0, _WPAD - _VOCAB), (0, 0)))
    wshift = jax.vmap(lambda w, s: jnp.roll(w, s, axis=0))(wpad, offs)

    return pl.pallas_call(
        _body,
        grid=(_BATCH // _TILE_B,),
        in_specs=[
            pl.BlockSpec((_TILE_B, _TOTAL_IN), lambda i: (i, 0)),
            pl.BlockSpec((_N_CAT, _WPAD, _EMB), lambda i: (0, 0, 0)),
        ],
        out_specs=pl.BlockSpec((_TILE_B, _TOTAL_OUT), lambda i: (i, 0)),
        out_shape=jax.ShapeDtypeStruct((_BATCH, _TOTAL_OUT), jnp.float32),
    )(inputs, wshift)


# speedup vs baseline: 1.5422x; 1.5422x over previous
import jax
import jax.numpy as jnp
from jax.experimental import pallas as pl
from jax.experimental.pallas import tpu as pltpu

_BATCH = 1024
_N_CONT = 13
_N_CAT = 26
_VOCAB = 1000
_EMB = 50
_TOTAL_IN = _N_CONT + _N_CAT * _VOCAB
_TOTAL_OUT = _N_CONT + _N_CAT * _EMB
_TILE_B = 128
_WPAD = 1152

_STARTS = [_N_CONT + f * _VOCAB for f in range(_N_CAT)]
_ALIGNED = [(s // 128) * 128 for s in _STARTS]
_OFFS = [s - a for s, a in zip(_STARTS, _ALIGNED)]


def _body(x_ref, w_ref, o_ref):
    o_ref[:, :_N_CONT] = x_ref[:, :_N_CONT]
    for f in range(_N_CAT):
        a = _ALIGNED[f]
        w = min(_WPAD, _TOTAL_IN - a)
        x = x_ref[:, a : a + w]
        o_ref[:, _N_CONT + f * _EMB : _N_CONT + (f + 1) * _EMB] = jnp.dot(
            x,
            w_ref[f, :w, :],
            preferred_element_type=jnp.float32,
            precision=jax.lax.Precision.DEFAULT,
        )


def kernel(inputs, embeddings):
    wshift = jnp.stack(
        [
            jnp.pad(embeddings[f], ((off, _WPAD - _VOCAB - off), (0, 0)))
            for f, off in enumerate(_OFFS)
        ]
    )

    return pl.pallas_call(
        _body,
        grid=(_BATCH // _TILE_B,),
        in_specs=[
            pl.BlockSpec((_TILE_B, _TOTAL_IN), lambda i: (i, 0)),
            pl.BlockSpec((_N_CAT, _WPAD, _EMB), lambda i: (0, 0, 0)),
        ],
        out_specs=pl.BlockSpec((_TILE_B, _TOTAL_OUT), lambda i: (i, 0)),
        out_shape=jax.ShapeDtypeStruct((_BATCH, _TOTAL_OUT), jnp.float32),
    )(inputs, wshift)
